# Initial kernel scaffold; baseline (speedup 1.0000x reference)
import jax
import jax.numpy as jnp
from jax.experimental import pallas as pl


def kernel(x, edge_index, Wl1, Wr1, b1, Wl2, Wr2, b2, Wlin1, blin1, Wlin2, blin2):
    raise NotImplementedError("write your pallas kernel here")


# speedup vs baseline: 6.6392x; 6.6392x over previous
import functools

import jax
import jax.numpy as jnp
from jax import lax
from jax.experimental import pallas as pl
from jax.experimental.pallas import tpu as pltpu
from jax.experimental.pallas import tpu_sc as plsc

_N = 10000
_E = 320000
_D = 128
_DA = 144
_DO = 40
_NC = 2
_NS = 16
_NW = _NC * _NS
_EPT = _E // _NW
_CH = 80
_NCHUNK = _EPT // _CH
_NPAD = 10240
_RPT = _NPAD // _NS


def _sc_agg_body(x_hbm, src_hbm, dst_hbm, zrow_hbm, agg_out,
                 src_v, dst_v, rows_v, agg_sh, sem):
    c = lax.axis_index("c")
    s = lax.axis_index("s")
    wid = c * _NS + s

    pltpu.sync_copy(src_hbm.at[wid], src_v)
    pltpu.sync_copy(dst_hbm.at[wid], dst_v)
    pltpu.sync_copy(zrow_hbm, agg_sh.at[pl.ds(s * _RPT, _RPT)])
    plsc.subcore_barrier()

    def step(i, carry):
        pltpu.async_copy(x_hbm.at[src_v.at[i]], rows_v, sem).wait()
        pltpu.sync_copy(rows_v, agg_sh.at[dst_v.at[i]], add=True)
        return carry

    lax.fori_loop(0, _NCHUNK, step, 0)
    plsc.subcore_barrier()

    sl = pl.ds(s * _RPT, _RPT)
    pltpu.sync_copy(agg_sh.at[sl], agg_out.at[c, sl])


@functools.lru_cache(maxsize=None)
def _make_sc_agg():
    mesh = plsc.VectorSubcoreMesh(core_axis_name="c", subcore_axis_name="s",
                                  num_cores=_NC, num_subcores=_NS)
    return pl.kernel(
        _sc_agg_body,
        out_type=jax.ShapeDtypeStruct((_NC, _NPAD, _DA), jnp.float32),
        mesh=mesh,
        scratch_types=[
            pltpu.VMEM((_NCHUNK, _CH), jnp.int32),
            pltpu.VMEM((_NCHUNK, _CH), jnp.int32),
            pltpu.VMEM((_CH, _DA), jnp.float32),
            pltpu.VMEM_SHARED((_NPAD, _DA), jnp.float32),
            pltpu.SemaphoreType.DMA,
        ],
        compiler_params=pltpu.CompilerParams(use_tc_tiling_on_sc=False),
        name="sage_sc_agg",
    )


def _mean_from_parts(parts):
    p = parts[0] + parts[1]
    cnt = p[:, _D:_D + 1]
    inv = 1.0 / jnp.maximum(cnt, 1.0)
    return p[:, :_D] * inv


def _tc_layer_body(parts, x, wl, wr, b, out):
    agg = _mean_from_parts(parts)
    h = (jnp.dot(agg, wl[...], preferred_element_type=jnp.float32)
         + jnp.dot(x[...][:, :_D], wr[...], preferred_element_type=jnp.float32)
         + b[...])
    h = jnp.maximum(h, 0.0)
    aug = jnp.concatenate(
        [h, jnp.ones((h.shape[0], 1), jnp.float32),
         jnp.zeros((h.shape[0], _DA - _D - 1), jnp.float32)], axis=1)
    out[...] = aug


def _tc_head_body(parts, x, wl, wr, b, wlin1, blin1, wlin2, blin2, out):
    agg = _mean_from_parts(parts)
    h2 = (jnp.dot(agg, wl[...], preferred_element_type=jnp.float32)
          + jnp.dot(x[...][:, :_D], wr[...], preferred_element_type=jnp.float32)
          + b[...])
    h2 = jnp.maximum(h2, 0.0)
    h3 = jnp.maximum(
        jnp.dot(h2, wlin1[...], preferred_element_type=jnp.float32) + blin1[...],
        0.0)
    logits = jnp.dot(h3, wlin2[...], preferred_element_type=jnp.float32) + blin2[...]
    m = jnp.max(logits, axis=-1, keepdims=True)
    lse = jnp.log(jnp.sum(jnp.exp(logits - m), axis=-1, keepdims=True)) + m
    out[...] = logits - lse


_BLK = 2000


def _tc_layer(parts, x_aug, wl, wr, b):
    return pl.pallas_call(
        _tc_layer_body,
        grid=(_N // _BLK,),
        in_specs=[
            pl.BlockSpec((_NC, _BLK, _DA), lambda i: (0, i, 0)),
            pl.BlockSpec((_BLK, _DA), lambda i: (i, 0)),
            pl.BlockSpec((_D, _D), lambda i: (0, 0)),
            pl.BlockSpec((_D, _D), lambda i: (0, 0)),
            pl.BlockSpec((1, _D), lambda i: (0, 0)),
        ],
        out_specs=pl.BlockSpec((_BLK, _DA), lambda i: (i, 0)),
        out_shape=jax.ShapeDtypeStruct((_N, _DA), jnp.float32),
        name="sage_tc_layer",
    )(parts, x_aug, wl, wr, b.reshape(1, _D))


def _tc_head(parts, x_aug, wl, wr, b, wlin1, blin1, wlin2, blin2):
    return pl.pallas_call(
        _tc_head_body,
        grid=(_N // _BLK,),
        in_specs=[
            pl.BlockSpec((_NC, _BLK, _DA), lambda i: (0, i, 0)),
            pl.BlockSpec((_BLK, _DA), lambda i: (i, 0)),
            pl.BlockSpec((_D, _D), lambda i: (0, 0)),
            pl.BlockSpec((_D, _D), lambda i: (0, 0)),
            pl.BlockSpec((1, _D), lambda i: (0, 0)),
            pl.BlockSpec((_D, _D), lambda i: (0, 0)),
            pl.BlockSpec((1, _D), lambda i: (0, 0)),
            pl.BlockSpec((_D, _DO), lambda i: (0, 0)),
            pl.BlockSpec((1, _DO), lambda i: (0, 0)),
        ],
        out_specs=pl.BlockSpec((_BLK, _DO), lambda i: (i, 0)),
        out_shape=jax.ShapeDtypeStruct((_N, _DO), jnp.float32),
        name="sage_tc_head",
    )(parts, x_aug, wl, wr, b.reshape(1, _D),
      wlin1, blin1.reshape(1, _D), wlin2, blin2.reshape(1, _DO))


def kernel(x, edge_index, Wl1, Wr1, b1, Wl2, Wr2, b2, Wlin1, blin1, Wlin2, blin2):
    ei = edge_index.astype(jnp.int32)
    src = ei[0].reshape(_NW, _NCHUNK, _CH)
    dst = ei[1].reshape(_NW, _NCHUNK, _CH)
    zrow = jnp.zeros((_RPT, _DA), jnp.float32)
    x_aug = jnp.concatenate(
        [x, jnp.ones((_N, 1), jnp.float32),
         jnp.zeros((_N, _DA - _D - 1), jnp.float32)], axis=1)

    sc_agg = _make_sc_agg()
    agg1 = sc_agg(x_aug, src, dst, zrow)
    h1_aug = _tc_layer(agg1, x_aug, Wl1, Wr1, b1)
    agg2 = sc_agg(h1_aug, src, dst, zrow)
    return _tc_head(agg2, h1_aug, Wl2, Wr2, b2, Wlin1, blin1, Wlin2, blin2)
